# baseline (speedup 1.0000x reference)
import functools

import jax
import jax.numpy as jnp
from jax.experimental import pallas as pl
from jax.experimental.pallas import tpu as pltpu

EPS = 1e-20


def _fused_block(k_count, scal_ref, self_ref, neigh_ref, gu_ref,
                 attw_ref, nw_ref, sw_ref, out_ref, ew_ref):
    K = neigh_ref.shape[1]
    s = self_ref[...]
    q = jnp.dot(s, attw_ref[...], preferred_element_type=jnp.float32)

    cols = []
    for kk in range(K):
        nbk = neigh_ref[:, kk, :]
        cols.append(jnp.sum(nbk * q, axis=1, keepdims=True))
    ew = jnp.concatenate(cols, axis=1)
    ew_ref[...] = ew

    inv_t = 1.0 / scal_ref[0]
    top_k = scal_ref[1]
    g = -jnp.log(-jnp.log(gu_ref[...] + EPS) + EPS)
    v = (ew + g) * inv_t
    v = v - jnp.max(v, axis=-1, keepdims=True)
    e = jnp.exp(v)
    mv = e / jnp.sum(e, axis=-1, keepdims=True)

    cnt = jnp.zeros_like(mv)
    for kk in range(K):
        cnt = cnt + (mv[:, kk:kk + 1] > mv).astype(jnp.float32)
    mask = (cnt < top_k).astype(jnp.float32)
    support = mv * mask * (1.0 / k_count)

    nm = jnp.zeros_like(s)
    for kk in range(K):
        nm = nm + neigh_ref[:, kk, :] * support[:, kk:kk + 1]
    fn = jnp.dot(nm, nw_ref[...], preferred_element_type=jnp.float32)
    fs = jnp.dot(s, sw_ref[...], preferred_element_type=jnp.float32)
    out_ref[...] = jnp.maximum(fs + fn, 0.0)


def kernel(self_vecs, neigh_vecs, temperature, gumbel_u, att_weights,
           neigh_weights, self_weights, top_k):
    n, k, d = neigh_vecs.shape
    o = neigh_weights.shape[1]
    block = 200
    grid = (n // block,)

    scalars = jnp.stack([temperature.astype(jnp.float32),
                         jnp.asarray(top_k, jnp.float32)])
    body = functools.partial(_fused_block, float(k))
    out, ew = pl.pallas_call(
        body,
        grid=grid,
        in_specs=[
            pl.BlockSpec(memory_space=pltpu.SMEM),
            pl.BlockSpec((block, d), lambda i: (i, 0)),
            pl.BlockSpec((block, k, d), lambda i: (i, 0, 0)),
            pl.BlockSpec((block, k), lambda i: (i, 0)),
            pl.BlockSpec((d, d), lambda i: (0, 0)),
            pl.BlockSpec((d, o), lambda i: (0, 0)),
            pl.BlockSpec((d, o), lambda i: (0, 0)),
        ],
        out_specs=[
            pl.BlockSpec((block, o), lambda i: (i, 0)),
            pl.BlockSpec((block, k), lambda i: (i, 0)),
        ],
        out_shape=[
            jax.ShapeDtypeStruct((n, o), jnp.float32),
            jax.ShapeDtypeStruct((n, k), jnp.float32),
        ],
    )(scalars, self_vecs, neigh_vecs, gumbel_u,
      att_weights, neigh_weights, self_weights)
    return (out, ew)


# speedup vs baseline: 1.3254x; 1.3254x over previous
import functools

import jax
import jax.numpy as jnp
from jax.experimental import pallas as pl
from jax.experimental.pallas import tpu as pltpu

EPS = 1e-20


def _fused_block(k_count, scal_ref, self_ref, neigh_ref, gu_ref,
                 attw_ref, nw_ref, sw_ref, g_ref, h_ref, out_ref, ew_ref):
    K = int(k_count)
    s = self_ref[...]
    q = jnp.dot(s, attw_ref[...], preferred_element_type=jnp.float32)

    nb = neigh_ref[...]
    qt = jnp.concatenate([q] * K, axis=1)
    ew = jnp.dot(nb * qt, g_ref[...],
                 preferred_element_type=jnp.float32)
    ew_ref[...] = ew

    inv_t = 1.0 / scal_ref[0]
    top_k = scal_ref[1]
    g = -jnp.log(-jnp.log(gu_ref[...] + EPS) + EPS)
    v = (ew + g) * inv_t
    v = v - jnp.max(v, axis=-1, keepdims=True)
    e = jnp.exp(v)
    mv = e / jnp.sum(e, axis=-1, keepdims=True)

    cnt = jnp.zeros_like(mv)
    for kk in range(K):
        cnt = cnt + (mv[:, kk:kk + 1] > mv).astype(jnp.float32)
    mask = (cnt < top_k).astype(jnp.float32)
    support = mv * mask * (1.0 / k_count)

    st = jnp.dot(support, g_ref[...].T,
                 preferred_element_type=jnp.float32)
    nm = jnp.dot(nb * st, h_ref[...],
                 preferred_element_type=jnp.float32)
    fn = jnp.dot(nm, nw_ref[...], preferred_element_type=jnp.float32)
    fs = jnp.dot(s, sw_ref[...], preferred_element_type=jnp.float32)
    out_ref[...] = jnp.maximum(fs + fn, 0.0)


def kernel(self_vecs, neigh_vecs, temperature, gumbel_u, att_weights,
           neigh_weights, self_weights, top_k):
    n, k, d = neigh_vecs.shape
    o = neigh_weights.shape[1]
    block = 200
    grid = (n // block,)

    neigh2 = neigh_vecs.reshape(n, k * d)
    j = jnp.arange(k * d)
    gmat = (j[:, None] // d == jnp.arange(k)[None, :]).astype(jnp.float32)
    hmat = (j[:, None] % d == jnp.arange(d)[None, :]).astype(jnp.float32)

    scalars = jnp.stack([temperature.astype(jnp.float32),
                         jnp.asarray(top_k, jnp.float32)])
    body = functools.partial(_fused_block, float(k))
    out, ew = pl.pallas_call(
        body,
        grid=grid,
        in_specs=[
            pl.BlockSpec(memory_space=pltpu.SMEM),
            pl.BlockSpec((block, d), lambda i: (i, 0)),
            pl.BlockSpec((block, k * d), lambda i: (i, 0)),
            pl.BlockSpec((block, k), lambda i: (i, 0)),
            pl.BlockSpec((d, d), lambda i: (0, 0)),
            pl.BlockSpec((d, o), lambda i: (0, 0)),
            pl.BlockSpec((d, o), lambda i: (0, 0)),
            pl.BlockSpec((k * d, k), lambda i: (0, 0)),
            pl.BlockSpec((k * d, d), lambda i: (0, 0)),
        ],
        out_specs=[
            pl.BlockSpec((block, o), lambda i: (i, 0)),
            pl.BlockSpec((block, k), lambda i: (i, 0)),
        ],
        out_shape=[
            jax.ShapeDtypeStruct((n, o), jnp.float32),
            jax.ShapeDtypeStruct((n, k), jnp.float32),
        ],
    )(scalars, self_vecs, neigh2, gumbel_u,
      att_weights, neigh_weights, self_weights, gmat, hmat)
    return (out, ew)


# speedup vs baseline: 1.5986x; 1.2061x over previous
import functools

import jax
import jax.numpy as jnp
from jax.experimental import pallas as pl
from jax.experimental.pallas import tpu as pltpu

EPS = 1e-20


def _fused_block(k_count, scal_ref, self_ref, neigh_ref, gu_ref,
                 attw_ref, nw_ref, sw_ref, g_ref, h_ref, out_ref, ew_ref):
    K = int(k_count)
    s = self_ref[...]
    q = jnp.dot(s, attw_ref[...], preferred_element_type=jnp.float32)

    B = self_ref.shape[0]
    D = self_ref.shape[1]
    nb = neigh_ref[...].reshape(B, K * D)
    qt = jnp.concatenate([q] * K, axis=1)
    ew = jnp.dot(nb * qt, g_ref[...],
                 preferred_element_type=jnp.float32)
    ew_ref[...] = ew

    inv_t = 1.0 / scal_ref[0]
    top_k = scal_ref[1]
    g = -jnp.log(-jnp.log(gu_ref[...] + EPS) + EPS)
    v = (ew + g) * inv_t
    v = v - jnp.max(v, axis=-1, keepdims=True)
    e = jnp.exp(v)
    mv = e / jnp.sum(e, axis=-1, keepdims=True)

    cnt = jnp.zeros_like(mv)
    for kk in range(K):
        cnt = cnt + (mv[:, kk:kk + 1] > mv).astype(jnp.float32)
    mask = (cnt < top_k).astype(jnp.float32)
    support = mv * mask * (1.0 / k_count)

    st = jnp.dot(support, g_ref[...].T,
                 preferred_element_type=jnp.float32)
    nm = jnp.dot(nb * st, h_ref[...],
                 preferred_element_type=jnp.float32)
    fn = jnp.dot(nm, nw_ref[...], preferred_element_type=jnp.float32)
    fs = jnp.dot(s, sw_ref[...], preferred_element_type=jnp.float32)
    out_ref[...] = jnp.maximum(fs + fn, 0.0)


def kernel(self_vecs, neigh_vecs, temperature, gumbel_u, att_weights,
           neigh_weights, self_weights, top_k):
    n, k, d = neigh_vecs.shape
    o = neigh_weights.shape[1]
    block = 200
    grid = (n // block,)

    neigh2 = neigh_vecs.reshape(n * k, d)
    j = jnp.arange(k * d)
    gmat = (j[:, None] // d == jnp.arange(k)[None, :]).astype(jnp.float32)
    hmat = (j[:, None] % d == jnp.arange(d)[None, :]).astype(jnp.float32)

    scalars = jnp.stack([temperature.astype(jnp.float32),
                         jnp.asarray(top_k, jnp.float32)])
    body = functools.partial(_fused_block, float(k))
    out, ew = pl.pallas_call(
        body,
        grid=grid,
        in_specs=[
            pl.BlockSpec(memory_space=pltpu.SMEM),
            pl.BlockSpec((block, d), lambda i: (i, 0)),
            pl.BlockSpec((block * k, d), lambda i: (i, 0)),
            pl.BlockSpec((block, k), lambda i: (i, 0)),
            pl.BlockSpec((d, d), lambda i: (0, 0)),
            pl.BlockSpec((d, o), lambda i: (0, 0)),
            pl.BlockSpec((d, o), lambda i: (0, 0)),
            pl.BlockSpec((k * d, k), lambda i: (0, 0)),
            pl.BlockSpec((k * d, d), lambda i: (0, 0)),
        ],
        out_specs=[
            pl.BlockSpec((block, o), lambda i: (i, 0)),
            pl.BlockSpec((block, k), lambda i: (i, 0)),
        ],
        out_shape=[
            jax.ShapeDtypeStruct((n, o), jnp.float32),
            jax.ShapeDtypeStruct((n, k), jnp.float32),
        ],
    )(scalars, self_vecs, neigh2, gumbel_u,
      att_weights, neigh_weights, self_weights, gmat, hmat)
    return (out, ew)


# speedup vs baseline: 2.7527x; 1.7219x over previous
import functools

import jax
import jax.numpy as jnp
from jax.experimental import pallas as pl
from jax.experimental.pallas import tpu as pltpu

EPS = 1e-20


def _make_copy(neigh_hbm, scratch, sems, step, slot, kk, blk, d):
    return pltpu.make_async_copy(
        neigh_hbm.at[pl.ds(step * blk, blk), kk, :],
        scratch.at[slot, :, pl.ds(kk * d, d)],
        sems.at[slot, kk])


def _fused_block(k_count, blk, scal_ref, self_ref, neigh_hbm, gu_ref,
                 attw_ref, nw_ref, sw_ref, g_ref, gt_ref, h_ref,
                 out_ref, ew_ref, scratch, sems):
    K = int(k_count)
    D = self_ref.shape[1]
    i = pl.program_id(0)
    nsteps = pl.num_programs(0)

    @pl.when(i == 0)
    def _():
        for kk in range(K):
            _make_copy(neigh_hbm, scratch, sems, i, 0, kk, blk, D).start()

    @pl.when(i + 1 < nsteps)
    def _():
        for kk in range(K):
            _make_copy(neigh_hbm, scratch, sems, i + 1, (i + 1) % 2, kk,
                       blk, D).start()

    slot = i % 2
    for kk in range(K):
        _make_copy(neigh_hbm, scratch, sems, i, slot, kk, blk, D).wait()

    s = self_ref[...]
    q = jnp.dot(s, attw_ref[...], preferred_element_type=jnp.float32)

    nb = scratch[slot]
    qt = jnp.concatenate([q] * K, axis=1)
    ew = jnp.dot(nb * qt, g_ref[...],
                 preferred_element_type=jnp.float32)
    ew_ref[...] = ew

    inv_t = 1.0 / scal_ref[0]
    top_k = scal_ref[1]
    g = -jnp.log(-jnp.log(gu_ref[...] + EPS) + EPS)
    v = (ew + g) * inv_t
    v = v - jnp.max(v, axis=-1, keepdims=True)
    e = jnp.exp(v)
    mv = e / jnp.sum(e, axis=-1, keepdims=True)

    mvt = mv.T
    cntt = jnp.zeros_like(mvt)
    for kk in range(K):
        cntt = cntt + (mvt[kk:kk + 1, :] > mvt).astype(jnp.float32)
    maskt = (cntt < top_k).astype(jnp.float32)
    support = (mvt * maskt).T * (1.0 / k_count)

    st = jnp.dot(support, gt_ref[...],
                 preferred_element_type=jnp.float32)
    nm = jnp.dot(nb * st, h_ref[...],
                 preferred_element_type=jnp.float32)
    fn = jnp.dot(nm, nw_ref[...], preferred_element_type=jnp.float32)
    fs = jnp.dot(s, sw_ref[...], preferred_element_type=jnp.float32)
    out_ref[...] = jnp.maximum(fs + fn, 0.0)


def kernel(self_vecs, neigh_vecs, temperature, gumbel_u, att_weights,
           neigh_weights, self_weights, top_k):
    n, k, d = neigh_vecs.shape
    o = neigh_weights.shape[1]
    block = 200
    grid = (n // block,)

    j = jnp.arange(k * d)
    gmat = (j[:, None] // d == jnp.arange(k)[None, :]).astype(jnp.float32)
    hmat = (j[:, None] % d == jnp.arange(d)[None, :]).astype(jnp.float32)

    scalars = jnp.stack([temperature.astype(jnp.float32),
                         jnp.asarray(top_k, jnp.float32)])
    body = functools.partial(_fused_block, float(k), block)
    out, ew = pl.pallas_call(
        body,
        grid=grid,
        in_specs=[
            pl.BlockSpec(memory_space=pltpu.SMEM),
            pl.BlockSpec((block, d), lambda i: (i, 0)),
            pl.BlockSpec(memory_space=pl.ANY),
            pl.BlockSpec((block, k), lambda i: (i, 0)),
            pl.BlockSpec((d, d), lambda i: (0, 0)),
            pl.BlockSpec((d, o), lambda i: (0, 0)),
            pl.BlockSpec((d, o), lambda i: (0, 0)),
            pl.BlockSpec((k * d, k), lambda i: (0, 0)),
            pl.BlockSpec((k, k * d), lambda i: (0, 0)),
            pl.BlockSpec((k * d, d), lambda i: (0, 0)),
        ],
        out_specs=[
            pl.BlockSpec((block, o), lambda i: (i, 0)),
            pl.BlockSpec((block, k), lambda i: (i, 0)),
        ],
        out_shape=[
            jax.ShapeDtypeStruct((n, o), jnp.float32),
            jax.ShapeDtypeStruct((n, k), jnp.float32),
        ],
        scratch_shapes=[
            pltpu.VMEM((2, block, k * d), jnp.float32),
            pltpu.SemaphoreType.DMA((2, k)),
        ],
    )(scalars, self_vecs, neigh_vecs, gumbel_u,
      att_weights, neigh_weights, self_weights, gmat, gmat.T, hmat)
    return (out, ew)


# speedup vs baseline: 3.5436x; 1.2874x over previous
import functools

import jax
import jax.numpy as jnp
from jax.experimental import pallas as pl
from jax.experimental.pallas import tpu as pltpu

EPS = 1e-20


def _make_copy(neigh_hbm, scratch, sems, step, slot, kk, blk, d):
    return pltpu.make_async_copy(
        neigh_hbm.at[pl.ds(step * blk, blk), kk, :],
        scratch.at[slot, :, pl.ds(kk * d, d)],
        sems.at[slot, kk])


def _fused_block(k_count, blk, scal_ref, self_ref, neigh_hbm, gu_ref,
                 attw_ref, nw_ref, sw_ref, g_ref, gt_ref, h_ref,
                 out_ref, ew_ref, scratch, sems):
    K = int(k_count)
    D = self_ref.shape[1]
    i = pl.program_id(0)
    nsteps = pl.num_programs(0)

    @pl.when(i == 0)
    def _():
        for kk in range(K):
            _make_copy(neigh_hbm, scratch, sems, i, 0, kk, blk, D).start()

    @pl.when(i + 1 < nsteps)
    def _():
        for kk in range(K):
            _make_copy(neigh_hbm, scratch, sems, i + 1, (i + 1) % 2, kk,
                       blk, D).start()

    slot = i % 2
    for kk in range(K):
        _make_copy(neigh_hbm, scratch, sems, i, slot, kk, blk, D).wait()

    s = self_ref[...]
    q = jnp.dot(s, attw_ref[...], preferred_element_type=jnp.float32)

    nb = scratch[slot]
    qt = jnp.concatenate([q] * K, axis=1)
    ew = jnp.dot(nb * qt, g_ref[...],
                 preferred_element_type=jnp.float32)
    ew_ref[...] = ew

    inv_t = 1.0 / scal_ref[0]
    top_k = scal_ref[1]
    g = -jnp.log(-jnp.log(gu_ref[...] + EPS) + EPS)
    v = (ew + g) * inv_t
    v = v - jnp.max(v, axis=-1, keepdims=True)
    e = jnp.exp(v)
    mv = e / jnp.sum(e, axis=-1, keepdims=True)

    mvt = mv.T
    cntt = jnp.zeros_like(mvt)
    for kk in range(K):
        cntt = cntt + (mvt[kk:kk + 1, :] > mvt).astype(jnp.float32)
    maskt = (cntt < top_k).astype(jnp.float32)
    support = (mvt * maskt).T * (1.0 / k_count)

    st = jnp.dot(support, gt_ref[...],
                 preferred_element_type=jnp.float32)
    nm = jnp.dot(nb * st, h_ref[...],
                 preferred_element_type=jnp.float32)
    fn = jnp.dot(nm, nw_ref[...], preferred_element_type=jnp.float32)
    fs = jnp.dot(s, sw_ref[...], preferred_element_type=jnp.float32)
    out_ref[...] = jnp.maximum(fs + fn, 0.0)


def kernel(self_vecs, neigh_vecs, temperature, gumbel_u, att_weights,
           neigh_weights, self_weights, top_k):
    n, k, d = neigh_vecs.shape
    o = neigh_weights.shape[1]
    block = 400
    grid = (n // block,)

    j = jnp.arange(k * d)
    gmat = (j[:, None] // d == jnp.arange(k)[None, :]).astype(jnp.float32)
    hmat = (j[:, None] % d == jnp.arange(d)[None, :]).astype(jnp.float32)

    scalars = jnp.stack([temperature.astype(jnp.float32),
                         jnp.asarray(top_k, jnp.float32)])
    body = functools.partial(_fused_block, float(k), block)
    out, ew = pl.pallas_call(
        body,
        grid=grid,
        in_specs=[
            pl.BlockSpec(memory_space=pltpu.SMEM),
            pl.BlockSpec((block, d), lambda i: (i, 0)),
            pl.BlockSpec(memory_space=pl.ANY),
            pl.BlockSpec((block, k), lambda i: (i, 0)),
            pl.BlockSpec((d, d), lambda i: (0, 0)),
            pl.BlockSpec((d, o), lambda i: (0, 0)),
            pl.BlockSpec((d, o), lambda i: (0, 0)),
            pl.BlockSpec((k * d, k), lambda i: (0, 0)),
            pl.BlockSpec((k, k * d), lambda i: (0, 0)),
            pl.BlockSpec((k * d, d), lambda i: (0, 0)),
        ],
        out_specs=[
            pl.BlockSpec((block, o), lambda i: (i, 0)),
            pl.BlockSpec((block, k), lambda i: (i, 0)),
        ],
        out_shape=[
            jax.ShapeDtypeStruct((n, o), jnp.float32),
            jax.ShapeDtypeStruct((n, k), jnp.float32),
        ],
        scratch_shapes=[
            pltpu.VMEM((2, block, k * d), jnp.float32),
            pltpu.SemaphoreType.DMA((2, k)),
        ],
    )(scalars, self_vecs, neigh_vecs, gumbel_u,
      att_weights, neigh_weights, self_weights, gmat, gmat.T, hmat)
    return (out, ew)


# speedup vs baseline: 3.8190x; 1.0777x over previous
import functools

import jax
import jax.numpy as jnp
from jax.experimental import pallas as pl
from jax.experimental.pallas import tpu as pltpu

EPS = 1e-20


def _make_copy(neigh_hbm, scratch, sems, step, slot, kk, blk, d):
    return pltpu.make_async_copy(
        neigh_hbm.at[pl.ds(step * blk, blk), kk, :],
        scratch.at[slot, :, pl.ds(kk * d, d)],
        sems.at[slot, kk])


def _fused_block(k_count, blk, scal_ref, self_ref, neigh_hbm, gu_ref,
                 attw_ref, nw_ref, sw_ref, g_ref, gt_ref, h_ref,
                 out_ref, ew_ref, scratch, sems):
    K = int(k_count)
    D = self_ref.shape[1]
    i = pl.program_id(0)
    nsteps = pl.num_programs(0)

    @pl.when(i == 0)
    def _():
        for kk in range(K):
            _make_copy(neigh_hbm, scratch, sems, i, 0, kk, blk, D).start()

    @pl.when(i + 1 < nsteps)
    def _():
        for kk in range(K):
            _make_copy(neigh_hbm, scratch, sems, i + 1, (i + 1) % 2, kk,
                       blk, D).start()

    slot = i % 2
    for kk in range(K):
        _make_copy(neigh_hbm, scratch, sems, i, slot, kk, blk, D).wait()

    s = self_ref[...]
    q = jnp.dot(s, attw_ref[...], preferred_element_type=jnp.float32)

    nb = scratch[slot]
    qt = jnp.concatenate([q] * K, axis=1)
    ew = jnp.dot(nb * qt, g_ref[...],
                 preferred_element_type=jnp.float32)
    ew_ref[...] = ew

    inv_t = 1.0 / scal_ref[0]
    top_k = scal_ref[1]
    g = -jnp.log(-jnp.log(gu_ref[...] + EPS) + EPS)
    v = (ew + g) * inv_t
    v = v - jnp.max(v, axis=-1, keepdims=True)
    e = jnp.exp(v)
    mv = e / jnp.sum(e, axis=-1, keepdims=True)

    mvt = mv.T
    cntt = jnp.zeros_like(mvt)
    for kk in range(K):
        cntt = cntt + (mvt[kk:kk + 1, :] > mvt).astype(jnp.float32)
    maskt = (cntt < top_k).astype(jnp.float32)
    support = (mvt * maskt).T * (1.0 / k_count)

    st = jnp.dot(support, gt_ref[...],
                 preferred_element_type=jnp.float32)
    nm = jnp.dot(nb * st, h_ref[...],
                 preferred_element_type=jnp.float32)
    fn = jnp.dot(nm, nw_ref[...], preferred_element_type=jnp.float32)
    fs = jnp.dot(s, sw_ref[...], preferred_element_type=jnp.float32)
    out_ref[...] = jnp.maximum(fs + fn, 0.0)


def kernel(self_vecs, neigh_vecs, temperature, gumbel_u, att_weights,
           neigh_weights, self_weights, top_k):
    n, k, d = neigh_vecs.shape
    o = neigh_weights.shape[1]
    block = 1000
    grid = (n // block,)

    j = jnp.arange(k * d)
    gmat = (j[:, None] // d == jnp.arange(k)[None, :]).astype(jnp.float32)
    hmat = (j[:, None] % d == jnp.arange(d)[None, :]).astype(jnp.float32)

    scalars = jnp.stack([temperature.astype(jnp.float32),
                         jnp.asarray(top_k, jnp.float32)])
    body = functools.partial(_fused_block, float(k), block)
    out, ew = pl.pallas_call(
        body,
        grid=grid,
        in_specs=[
            pl.BlockSpec(memory_space=pltpu.SMEM),
            pl.BlockSpec((block, d), lambda i: (i, 0)),
            pl.BlockSpec(memory_space=pl.ANY),
            pl.BlockSpec((block, k), lambda i: (i, 0)),
            pl.BlockSpec((d, d), lambda i: (0, 0)),
            pl.BlockSpec((d, o), lambda i: (0, 0)),
            pl.BlockSpec((d, o), lambda i: (0, 0)),
            pl.BlockSpec((k * d, k), lambda i: (0, 0)),
            pl.BlockSpec((k, k * d), lambda i: (0, 0)),
            pl.BlockSpec((k * d, d), lambda i: (0, 0)),
        ],
        out_specs=[
            pl.BlockSpec((block, o), lambda i: (i, 0)),
            pl.BlockSpec((block, k), lambda i: (i, 0)),
        ],
        out_shape=[
            jax.ShapeDtypeStruct((n, o), jnp.float32),
            jax.ShapeDtypeStruct((n, k), jnp.float32),
        ],
        scratch_shapes=[
            pltpu.VMEM((2, block, k * d), jnp.float32),
            pltpu.SemaphoreType.DMA((2, k)),
        ],
    )(scalars, self_vecs, neigh_vecs, gumbel_u,
      att_weights, neigh_weights, self_weights, gmat, gmat.T, hmat)
    return (out, ew)


# speedup vs baseline: 4.0073x; 1.0493x over previous
import functools

import jax
import jax.numpy as jnp
import numpy as np
from jax.experimental import pallas as pl
from jax.experimental.pallas import tpu as pltpu

EPS = 1e-20


def _make_copy(neigh_hbm, scratch, sems, step, slot, kk, blk, d):
    return pltpu.make_async_copy(
        neigh_hbm.at[pl.ds(step * blk, blk), kk, :],
        scratch.at[slot, :, pl.ds(kk * d, d)],
        sems.at[slot, kk])


def _fused_block(k_count, blk, scal_ref, self_ref, neigh_hbm, gu_ref,
                 attw_ref, nw_ref, sw_ref, g_ref, gt_ref, h_ref,
                 out_ref, ew_ref, scratch, sems):
    K = int(k_count)
    D = self_ref.shape[1]
    i = pl.program_id(0)
    nsteps = pl.num_programs(0)

    @pl.when(i == 0)
    def _():
        for kk in range(K):
            _make_copy(neigh_hbm, scratch, sems, i, 0, kk, blk, D).start()

    @pl.when(i + 1 < nsteps)
    def _():
        for kk in range(K):
            _make_copy(neigh_hbm, scratch, sems, i + 1, (i + 1) % 2, kk,
                       blk, D).start()

    slot = i % 2
    for kk in range(K):
        _make_copy(neigh_hbm, scratch, sems, i, slot, kk, blk, D).wait()

    s = self_ref[...]
    q = jnp.dot(s, attw_ref[...], preferred_element_type=jnp.float32)

    nb = scratch[slot]
    qt = jnp.concatenate([q] * K, axis=1)
    ew = jnp.dot(nb * qt, g_ref[...],
                 preferred_element_type=jnp.float32)
    ew_ref[...] = ew

    inv_t = 1.0 / scal_ref[0]
    top_k = scal_ref[1]
    g = -jnp.log(-jnp.log(gu_ref[...] + EPS) + EPS)
    v = (ew + g) * inv_t
    v = v - jnp.max(v, axis=-1, keepdims=True)
    e = jnp.exp(v)
    mv = e / jnp.sum(e, axis=-1, keepdims=True)

    mvt = mv.T
    cntt = jnp.zeros_like(mvt)
    for kk in range(K):
        cntt = cntt + (mvt[kk:kk + 1, :] > mvt).astype(jnp.float32)
    maskt = (cntt < top_k).astype(jnp.float32)
    support = (mvt * maskt).T * (1.0 / k_count)

    st = jnp.dot(support, gt_ref[...],
                 preferred_element_type=jnp.float32)
    nm = jnp.dot(nb * st, h_ref[...],
                 preferred_element_type=jnp.float32)
    fn = jnp.dot(nm, nw_ref[...], preferred_element_type=jnp.float32)
    fs = jnp.dot(s, sw_ref[...], preferred_element_type=jnp.float32)
    out_ref[...] = jnp.maximum(fs + fn, 0.0)


def kernel(self_vecs, neigh_vecs, temperature, gumbel_u, att_weights,
           neigh_weights, self_weights, top_k):
    n, k, d = neigh_vecs.shape
    o = neigh_weights.shape[1]
    block = 1000
    grid = (n // block,)

    j = np.arange(k * d)
    gmat = jnp.asarray(j[:, None] // d == np.arange(k)[None, :],
                       dtype=jnp.float32)
    gmat_t = jnp.asarray(j[None, :] // d == np.arange(k)[:, None],
                         dtype=jnp.float32)
    hmat = jnp.asarray(j[:, None] % d == np.arange(d)[None, :],
                       dtype=jnp.float32)

    scalars = jnp.stack([temperature.astype(jnp.float32),
                         jnp.asarray(top_k, jnp.float32)])
    body = functools.partial(_fused_block, float(k), block)
    out, ew = pl.pallas_call(
        body,
        grid=grid,
        in_specs=[
            pl.BlockSpec(memory_space=pltpu.SMEM),
            pl.BlockSpec((block, d), lambda i: (i, 0)),
            pl.BlockSpec(memory_space=pl.ANY),
            pl.BlockSpec((block, k), lambda i: (i, 0)),
            pl.BlockSpec((d, d), lambda i: (0, 0)),
            pl.BlockSpec((d, o), lambda i: (0, 0)),
            pl.BlockSpec((d, o), lambda i: (0, 0)),
            pl.BlockSpec((k * d, k), lambda i: (0, 0)),
            pl.BlockSpec((k, k * d), lambda i: (0, 0)),
            pl.BlockSpec((k * d, d), lambda i: (0, 0)),
        ],
        out_specs=[
            pl.BlockSpec((block, o), lambda i: (i, 0)),
            pl.BlockSpec((block, k), lambda i: (i, 0)),
        ],
        out_shape=[
            jax.ShapeDtypeStruct((n, o), jnp.float32),
            jax.ShapeDtypeStruct((n, k), jnp.float32),
        ],
        scratch_shapes=[
            pltpu.VMEM((2, block, k * d), jnp.float32),
            pltpu.SemaphoreType.DMA((2, k)),
        ],
    )(scalars, self_vecs, neigh_vecs, gumbel_u,
      att_weights, neigh_weights, self_weights, gmat, gmat_t, hmat)
    return (out, ew)


# speedup vs baseline: 4.0354x; 1.0070x over previous
import functools

import jax
import jax.numpy as jnp
import numpy as np
from jax.experimental import pallas as pl
from jax.experimental.pallas import tpu as pltpu

EPS = 1e-20


def _make_copy(neigh_hbm, scratch, sems, step, slot, kk, blk, d):
    return pltpu.make_async_copy(
        neigh_hbm.at[pl.ds(step * blk, blk), kk, :],
        scratch.at[slot, :, pl.ds(kk * d, d)],
        sems.at[slot, kk])


def _fused_block(k_count, blk, temp_ref, topk_ref, self_ref, neigh_hbm,
                 gu_ref, attw_ref, nw_ref, sw_ref, g_ref, gt_ref, h_ref,
                 out_ref, ew_ref, scratch, sems):
    K = int(k_count)
    D = self_ref.shape[1]
    i = pl.program_id(0)
    nsteps = pl.num_programs(0)

    @pl.when(i == 0)
    def _():
        for kk in range(K):
            _make_copy(neigh_hbm, scratch, sems, i, 0, kk, blk, D).start()

    @pl.when(i + 1 < nsteps)
    def _():
        for kk in range(K):
            _make_copy(neigh_hbm, scratch, sems, i + 1, (i + 1) % 2, kk,
                       blk, D).start()

    slot = i % 2
    for kk in range(K):
        _make_copy(neigh_hbm, scratch, sems, i, slot, kk, blk, D).wait()

    s = self_ref[...]
    q = jnp.dot(s, attw_ref[...], preferred_element_type=jnp.float32)

    nb = scratch[slot]
    qt = jnp.concatenate([q] * K, axis=1)
    ew = jnp.dot(nb * qt, g_ref[...],
                 preferred_element_type=jnp.float32)
    ew_ref[...] = ew

    inv_t = 1.0 / temp_ref[0]
    top_k = topk_ref[0].astype(jnp.float32)
    g = -jnp.log(-jnp.log(gu_ref[...] + EPS) + EPS)
    v = (ew + g) * inv_t
    v = v - jnp.max(v, axis=-1, keepdims=True)
    e = jnp.exp(v)
    mv = e / jnp.sum(e, axis=-1, keepdims=True)

    mvt = mv.T
    cntt = jnp.zeros_like(mvt)
    for kk in range(K):
        cntt = cntt + (mvt[kk:kk + 1, :] > mvt).astype(jnp.float32)
    maskt = (cntt < top_k).astype(jnp.float32)
    support = (mvt * maskt).T * (1.0 / k_count)

    st = jnp.dot(support, gt_ref[...],
                 preferred_element_type=jnp.float32)
    nm = jnp.dot(nb * st, h_ref[...],
                 preferred_element_type=jnp.float32)
    fn = jnp.dot(nm, nw_ref[...], preferred_element_type=jnp.float32)
    fs = jnp.dot(s, sw_ref[...], preferred_element_type=jnp.float32)
    out_ref[...] = jnp.maximum(fs + fn, 0.0)


def kernel(self_vecs, neigh_vecs, temperature, gumbel_u, att_weights,
           neigh_weights, self_weights, top_k):
    n, k, d = neigh_vecs.shape
    o = neigh_weights.shape[1]
    block = 1000
    grid = (n // block,)

    j = np.arange(k * d)
    gmat = jnp.asarray(j[:, None] // d == np.arange(k)[None, :],
                       dtype=jnp.float32)
    gmat_t = jnp.asarray(j[None, :] // d == np.arange(k)[:, None],
                         dtype=jnp.float32)
    hmat = jnp.asarray(j[:, None] % d == np.arange(d)[None, :],
                       dtype=jnp.float32)

    body = functools.partial(_fused_block, float(k), block)
    out, ew = pl.pallas_call(
        body,
        grid=grid,
        in_specs=[
            pl.BlockSpec(memory_space=pltpu.SMEM),
            pl.BlockSpec(memory_space=pltpu.SMEM),
            pl.BlockSpec((block, d), lambda i: (i, 0)),
            pl.BlockSpec(memory_space=pl.ANY),
            pl.BlockSpec((block, k), lambda i: (i, 0)),
            pl.BlockSpec((d, d), lambda i: (0, 0)),
            pl.BlockSpec((d, o), lambda i: (0, 0)),
            pl.BlockSpec((d, o), lambda i: (0, 0)),
            pl.BlockSpec((k * d, k), lambda i: (0, 0)),
            pl.BlockSpec((k, k * d), lambda i: (0, 0)),
            pl.BlockSpec((k * d, d), lambda i: (0, 0)),
        ],
        out_specs=[
            pl.BlockSpec((block, o), lambda i: (i, 0)),
            pl.BlockSpec((block, k), lambda i: (i, 0)),
        ],
        out_shape=[
            jax.ShapeDtypeStruct((n, o), jnp.float32),
            jax.ShapeDtypeStruct((n, k), jnp.float32),
        ],
        scratch_shapes=[
            pltpu.VMEM((2, block, k * d), jnp.float32),
            pltpu.SemaphoreType.DMA((2, k)),
        ],
    )(temperature.reshape(1), jnp.asarray(top_k).reshape(1),
      self_vecs, neigh_vecs, gumbel_u,
      att_weights, neigh_weights, self_weights, gmat, gmat_t, hmat)
    return (out, ew)
